# Initial kernel scaffold; baseline (speedup 1.0000x reference)
import jax
import jax.numpy as jnp
from jax.experimental import pallas as pl


def kernel(pts, MModel_params, motion_latent, v_template, shapedirs, W0, b0, W1, b1, W2, b2, W3, b3, W4, b4, W5, b5, W6, b6, W7, b7, Wout, bout):
    raise NotImplementedError("write your pallas kernel here")


# speedup vs baseline: 2.6723x; 2.6723x over previous
import numpy as np
import jax
import jax.numpy as jnp
from jax.experimental import pallas as pl

V = 5023
VP = 5120
VT = 512
PT = 1024
HID = 128


def _blend_body(vt_ref, sd_ref, co_ref, def_ref, shift_ref):
    sc = jax.lax.dot_general(sd_ref[...], co_ref[...],
                             (((1,), (0,)), ((), ())),
                             preferred_element_type=jnp.float32)
    def_ref[...] = (vt_ref[...] + sc) * 4.0
    shift_ref[...] = -4.0 * sc


def _main_body(pts_ref, mot_ref, defT_ref, shift_ref,
               W0_ref, W4a_ref, W4b_ref, Wm_ref, Wout_ref,
               bs_ref, bout_ref, out_ref, feat_ref):
    pts = pts_ref[...]
    psq = jnp.sum(pts * pts, axis=1, keepdims=True)

    rmin = jnp.full((PT, 1), jnp.inf, jnp.float32)
    rshift = jnp.zeros((PT, 3), jnp.float32)
    for j in range(VP // VT):
        dT = defT_ref[:, j * VT:(j + 1) * VT]
        st = shift_ref[j * VT:(j + 1) * VT, :]
        vsq = jnp.sum(dT * dT, axis=0, keepdims=True)
        dot = jax.lax.dot_general(pts, dT, (((1,), (0,)), ((), ())),
                                  preferred_element_type=jnp.float32)
        d2 = jnp.maximum(psq + vsq - 2.0 * dot, 0.0)
        lane = jax.lax.broadcasted_iota(jnp.int32, (PT, VT), 1)
        d2 = jnp.where(j * VT + lane < V, d2, jnp.inf)
        tmin = jnp.min(d2, axis=1, keepdims=True)
        cand = jnp.where(d2 == tmin, lane, VT)
        targ = jnp.min(cand, axis=1, keepdims=True)
        onehot = (lane == targ).astype(jnp.float32)
        tshift = jax.lax.dot_general(onehot, st, (((1,), (0,)), ((), ())),
                                     preferred_element_type=jnp.float32)
        upd = tmin < rmin
        rmin = jnp.where(upd, tmin, rmin)
        rshift = jnp.where(upd, tshift, rshift)

    weight = 1.0 / jnp.exp(rmin)
    shifts = rshift * weight

    pieces = [pts * (2.0 ** l) for l in range(10)] + [shifts, shifts * 2.0]
    F = jnp.concatenate(pieces, axis=1)
    feat0 = jnp.concatenate(
        [pts, shifts, jnp.sin(F), jnp.cos(F), mot_ref[...],
         jnp.zeros((PT, 18), jnp.float32)], axis=1)

    mm = lambda a, b: jax.lax.dot_general(
        a, b, (((1,), (0,)), ((), ())), preferred_element_type=jnp.float32)

    x = jnp.maximum(mm(feat0, W0_ref[...]) + bs_ref[0:1, :], 0.0)
    for li in range(3):
        x = jnp.maximum(mm(x, Wm_ref[li]) + bs_ref[li + 1:li + 2, :], 0.0)
    x = jnp.maximum(mm(feat0, W4a_ref[...]) + mm(x, W4b_ref[...])
                    + bs_ref[4:5, :], 0.0)
    for li in range(3, 6):
        x = mm(x, Wm_ref[li]) + bs_ref[li + 2:li + 3, :]
        last = x
        x = jnp.maximum(x, 0.0)
    feat_ref[...] = last
    mlp_out = mm(x, Wout_ref[...]) + bout_ref[...]
    out_ref[...] = pts + shifts + mlp_out


def _perm110():
    p = [0, 1, 2, 63, 64, 65]
    for l in range(10):
        p += [3 + 6 * l + c for c in range(3)]
    for l in range(2):
        p += [66 + 6 * l + c for c in range(3)]
    for l in range(10):
        p += [6 + 6 * l + c for c in range(3)]
    for l in range(2):
        p += [69 + 6 * l + c for c in range(3)]
    p += list(range(78, 110))
    return np.array(p)


_PERM = _perm110()


def _permpad(W):
    return jnp.zeros((128, HID), W.dtype).at[:110].set(W[_PERM])


def kernel(pts, MModel_params, motion_latent, v_template, shapedirs,
           W0, b0, W1, b1, W2, b2, W3, b3, W4, b4, W5, b5, W6, b6, W7, b7,
           Wout, bout):
    B, P, _ = pts.shape
    coeff = MModel_params[0, :150].reshape(150, 1)
    sd2 = shapedirs.reshape(V * 3, 150)
    vt2 = v_template.reshape(V * 3, 1)

    def_flat, shift_flat = pl.pallas_call(
        _blend_body,
        out_shape=[jax.ShapeDtypeStruct((V * 3, 1), jnp.float32),
                   jax.ShapeDtypeStruct((V * 3, 1), jnp.float32)],
    )(vt2, sd2, coeff)

    deformed = def_flat.reshape(V, 3)
    mesh_shift = shift_flat.reshape(V, 3)
    defT = jnp.pad(deformed, ((0, VP - V), (0, 0))).T
    shiftp = jnp.pad(mesh_shift, ((0, VP - V), (0, 0)))

    W0p = _permpad(W0)
    W4a = _permpad(W4[:110])
    W4b = W4[110:238]
    Wm = jnp.stack([W1, W2, W3, W5, W6, W7])
    bs = jnp.stack([b0, b1, b2, b3, b4, b5, b6, b7])
    bout2 = bout.reshape(1, 3)

    cst = lambda *blk: pl.BlockSpec(blk, lambda i: tuple(0 for _ in blk))
    out, feat = pl.pallas_call(
        _main_body,
        grid=(P // PT,),
        in_specs=[
            pl.BlockSpec((PT, 3), lambda i: (i, 0)),
            pl.BlockSpec((PT, 32), lambda i: (i, 0)),
            cst(3, VP),
            cst(VP, 3),
            cst(128, HID),
            cst(128, HID),
            cst(HID, HID),
            cst(6, HID, HID),
            cst(HID, 3),
            cst(8, HID),
            cst(1, 3),
        ],
        out_specs=[pl.BlockSpec((PT, 3), lambda i: (i, 0)),
                   pl.BlockSpec((PT, HID), lambda i: (i, 0))],
        out_shape=[jax.ShapeDtypeStruct((P, 3), jnp.float32),
                   jax.ShapeDtypeStruct((P, HID), jnp.float32)],
    )(pts[0], motion_latent[0], defT, shiftp,
      W0p, W4a, W4b, Wm, Wout, bs, bout2)

    return out[None], feat[None]


# speedup vs baseline: 3.1019x; 1.1608x over previous
import numpy as np
import jax
import jax.numpy as jnp
from jax.experimental import pallas as pl
from jax.experimental.pallas import tpu as pltpu

V = 5023
VP = 5120
VT = 512
PT = 1024
HID = 128
NT = VP // VT


def _blend_body(vt_ref, sd_ref, co_ref, def_ref, shift_ref):
    sc = jax.lax.dot_general(sd_ref[...], co_ref[...],
                             (((1,), (0,)), ((), ())),
                             preferred_element_type=jnp.float32)
    def_ref[...] = (vt_ref[...] + sc) * 4.0
    shift_ref[...] = -4.0 * sc


def _main_body(pts_ref, mot_ref, defT_ref, shift_ref, M6_ref,
               W0_ref, W4a_ref, W4b_ref, Wm_ref, Wout_ref,
               bs_ref, bout_ref, out_ref, feat_ref):
    dTf = defT_ref[...]
    vsqf = jnp.sum(dTf * dTf, axis=0, keepdims=True)

    pts = pts_ref[...]
    psq = jnp.sum(pts * pts, axis=1, keepdims=True)

    rmin = jnp.full((PT, 1), jnp.inf, jnp.float32)
    rshift = jnp.zeros((PT, 3), jnp.float32)
    lanef = jax.lax.broadcasted_iota(jnp.int32, (PT, VT), 1).astype(jnp.float32)
    for j in range(NT):
        sl = slice(j * VT, (j + 1) * VT)
        st = shift_ref[sl, :]
        dot = jax.lax.dot_general(pts, dTf[:, sl],
                                  (((1,), (0,)), ((), ())),
                                  preferred_element_type=jnp.float32)
        d2 = jnp.maximum(psq + vsqf[:, sl] - 2.0 * dot, 0.0)
        if j == NT - 1:
            d2 = jnp.where(lanef < float(V - j * VT), d2, jnp.inf)
        tmin = jnp.min(d2, axis=1, keepdims=True)
        cand = jnp.where(d2 == tmin, lanef, float(VT))
        targ = jnp.min(cand, axis=1, keepdims=True)
        onehot = (cand == targ).astype(jnp.float32)
        tshift = jax.lax.dot_general(onehot, st, (((1,), (0,)), ((), ())),
                                     preferred_element_type=jnp.float32)
        upd = tmin < rmin
        rmin = jnp.where(upd, tmin, rmin)
        rshift = jnp.where(upd, tshift, rshift)

    weight = 1.0 / jnp.exp(rmin)
    shifts = rshift * weight

    ps6 = jnp.concatenate([pts, shifts], axis=1)
    lin = jax.lax.dot_general(ps6, M6_ref[...], (((1,), (0,)), ((), ())),
                              preferred_element_type=jnp.float32,
                              precision=jax.lax.Precision.HIGHEST)
    s_all = jnp.sin(lin)
    c_all = jnp.cos(lin)
    lane = jax.lax.broadcasted_iota(jnp.int32, (PT, HID), 1)
    motp = jnp.concatenate([mot_ref[...],
                            jnp.zeros((PT, HID - 32), jnp.float32)], axis=1)
    feat0 = jnp.where(lane < 32, motp,
             jnp.where(lane < 38, lin,
              jnp.where(lane < 74, s_all,
               jnp.where(lane < 110, c_all, 0.0))))

    mm = lambda a, b: jax.lax.dot_general(
        a, b, (((1,), (0,)), ((), ())), preferred_element_type=jnp.float32)

    x = jnp.maximum(mm(feat0, W0_ref[...]) + bs_ref[0:1, :], 0.0)
    for li in range(3):
        x = jnp.maximum(mm(x, Wm_ref[li]) + bs_ref[li + 1:li + 2, :], 0.0)
    x = jnp.maximum(mm(feat0, W4a_ref[...]) + mm(x, W4b_ref[...])
                    + bs_ref[4:5, :], 0.0)
    for li in range(3, 6):
        x = mm(x, Wm_ref[li]) + bs_ref[li + 2:li + 3, :]
        last = x
        x = jnp.maximum(x, 0.0)
    feat_ref[...] = last
    mlp_out = mm(x, Wout_ref[...]) + bout_ref[...]
    out_ref[...] = pts + shifts + mlp_out


def _layouts():
    perm = []
    perm += list(range(78, 110))
    perm += [0, 1, 2, 63, 64, 65]
    for l in range(10):
        perm += [3 + 6 * l + c for c in range(3)]
    for l in range(2):
        perm += [66 + 6 * l + c for c in range(3)]
    for l in range(10):
        perm += [6 + 6 * l + c for c in range(3)]
    for l in range(2):
        perm += [69 + 6 * l + c for c in range(3)]
    M6 = np.zeros((6, HID), np.float32)
    for c in range(3):
        M6[c, 32 + c] = 1.0
        M6[3 + c, 35 + c] = 1.0
        for l in range(10):
            M6[c, 38 + 3 * l + c] = 2.0 ** l
            M6[c, 74 + 3 * l + c] = 2.0 ** l
        for l in range(2):
            M6[3 + c, 68 + 3 * l + c] = 2.0 ** l
            M6[3 + c, 104 + 3 * l + c] = 2.0 ** l
    return np.array(perm), M6


_PERM, _M6 = _layouts()


def _permpad(W):
    return jnp.zeros((HID, HID), W.dtype).at[:110].set(W[_PERM])


def kernel(pts, MModel_params, motion_latent, v_template, shapedirs,
           W0, b0, W1, b1, W2, b2, W3, b3, W4, b4, W5, b5, W6, b6, W7, b7,
           Wout, bout):
    B, P, _ = pts.shape
    coeff = MModel_params[0, :150].reshape(150, 1)
    sd2 = shapedirs.reshape(V * 3, 150)
    vt2 = v_template.reshape(V * 3, 1)

    def_flat, shift_flat = pl.pallas_call(
        _blend_body,
        out_shape=[jax.ShapeDtypeStruct((V * 3, 1), jnp.float32),
                   jax.ShapeDtypeStruct((V * 3, 1), jnp.float32)],
    )(vt2, sd2, coeff)

    deformed = def_flat.reshape(V, 3)
    mesh_shift = shift_flat.reshape(V, 3)
    defT = jnp.pad(deformed, ((0, VP - V), (0, 0))).T
    shiftp = jnp.pad(mesh_shift, ((0, VP - V), (0, 0)))

    W0p = _permpad(W0)
    W4a = _permpad(W4[:110])
    W4b = W4[110:238]
    Wm = jnp.stack([W1, W2, W3, W5, W6, W7])
    bs = jnp.stack([b0, b1, b2, b3, b4, b5, b6, b7])
    bout2 = bout.reshape(1, 3)
    M6 = jnp.asarray(_M6)

    cst = lambda *blk: pl.BlockSpec(blk, lambda i: tuple(0 for _ in blk))
    out, feat = pl.pallas_call(
        _main_body,
        grid=(P // PT,),
        in_specs=[
            pl.BlockSpec((PT, 3), lambda i: (i, 0)),
            pl.BlockSpec((PT, 32), lambda i: (i, 0)),
            cst(3, VP),
            cst(VP, 3),
            cst(6, HID),
            cst(HID, HID),
            cst(HID, HID),
            cst(HID, HID),
            cst(6, HID, HID),
            cst(HID, 3),
            cst(8, HID),
            cst(1, 3),
        ],
        out_specs=[pl.BlockSpec((PT, 3), lambda i: (i, 0)),
                   pl.BlockSpec((PT, HID), lambda i: (i, 0))],
        out_shape=[jax.ShapeDtypeStruct((P, 3), jnp.float32),
                   jax.ShapeDtypeStruct((P, HID), jnp.float32)],
    )(pts[0], motion_latent[0], defT, shiftp, M6,
      W0p, W4a, W4b, Wm, Wout, bs, bout2)

    return out[None], feat[None]


# speedup vs baseline: 3.1027x; 1.0002x over previous
import numpy as np
import jax
import jax.numpy as jnp
from jax.experimental import pallas as pl
from jax.experimental.pallas import tpu as pltpu

V = 5023
VP = 5120
VT = 512
PT = 1024
HID = 128
NT = VP // VT


def _blend_body(vt_ref, sd_ref, co_ref, def_ref, shift_ref):
    sc = jax.lax.dot_general(sd_ref[...], co_ref[...],
                             (((1,), (0,)), ((), ())),
                             preferred_element_type=jnp.float32)
    def_ref[...] = (vt_ref[...] + sc) * 4.0
    shift_ref[...] = -4.0 * sc


def _main_body(pts_ref, mot_ref, defT_ref, shift_ref, M6_ref,
               W0_ref, W4a_ref, W4b_ref, Wm_ref, Wout_ref,
               bs_ref, bout_ref, out_ref, feat_ref):
    dTf = defT_ref[...]
    vsqf = jnp.sum(dTf * dTf, axis=0, keepdims=True)

    pts = pts_ref[...]
    psq = jnp.sum(pts * pts, axis=1, keepdims=True)

    rmin = jnp.full((PT, 1), jnp.inf, jnp.float32)
    rshift = jnp.zeros((PT, 3), jnp.float32)
    lanef = jax.lax.broadcasted_iota(jnp.int32, (PT, VT), 1).astype(jnp.float32)
    for j in range(NT):
        sl = slice(j * VT, (j + 1) * VT)
        st = shift_ref[sl, :]
        dot = jax.lax.dot_general(pts, dTf[:, sl],
                                  (((1,), (0,)), ((), ())),
                                  preferred_element_type=jnp.float32)
        d2 = jnp.maximum(psq + vsqf[:, sl] - 2.0 * dot, 0.0)
        if j == NT - 1:
            d2 = jnp.where(lanef < float(V - j * VT), d2, jnp.inf)
        tmin = jnp.min(d2, axis=1, keepdims=True)
        cand = jnp.where(d2 == tmin, lanef, float(VT))
        targ = jnp.min(cand, axis=1, keepdims=True)
        onehot = (cand == targ).astype(jnp.float32)
        tshift = jax.lax.dot_general(onehot, st, (((1,), (0,)), ((), ())),
                                     preferred_element_type=jnp.float32)
        upd = tmin < rmin
        rmin = jnp.where(upd, tmin, rmin)
        rshift = jnp.where(upd, tshift, rshift)

    weight = 1.0 / jnp.exp(rmin)
    shifts = rshift * weight

    ps6 = jnp.concatenate([pts, shifts], axis=1)
    lin = jax.lax.dot_general(ps6, M6_ref[...], (((1,), (0,)), ((), ())),
                              preferred_element_type=jnp.float32,
                              precision=jax.lax.Precision.HIGHEST)
    s_all = jnp.sin(lin)
    c_all = jnp.cos(lin)
    lane = jax.lax.broadcasted_iota(jnp.int32, (PT, HID), 1)
    motp = jnp.concatenate([mot_ref[...],
                            jnp.zeros((PT, HID - 32), jnp.float32)], axis=1)
    feat0 = jnp.where(lane < 32, motp,
             jnp.where(lane < 38, lin,
              jnp.where(lane < 74, s_all,
               jnp.where(lane < 110, c_all, 0.0))))

    mm = lambda a, b: jax.lax.dot_general(
        a, b, (((1,), (0,)), ((), ())), preferred_element_type=jnp.float32)

    x = jnp.maximum(mm(feat0, W0_ref[...]) + bs_ref[0:1, :], 0.0)
    for li in range(3):
        x = jnp.maximum(mm(x, Wm_ref[li]) + bs_ref[li + 1:li + 2, :], 0.0)
    x = jnp.maximum(mm(feat0, W4a_ref[...]) + mm(x, W4b_ref[...])
                    + bs_ref[4:5, :], 0.0)
    for li in range(3, 6):
        x = mm(x, Wm_ref[li]) + bs_ref[li + 2:li + 3, :]
        last = x
        x = jnp.maximum(x, 0.0)
    feat_ref[...] = last
    mlp_out = mm(x, Wout_ref[...]) + bout_ref[...]
    out_ref[...] = pts + shifts + mlp_out


def _layouts():
    perm = []
    perm += list(range(78, 110))
    perm += [0, 1, 2, 63, 64, 65]
    for l in range(10):
        perm += [3 + 6 * l + c for c in range(3)]
    for l in range(2):
        perm += [66 + 6 * l + c for c in range(3)]
    for l in range(10):
        perm += [6 + 6 * l + c for c in range(3)]
    for l in range(2):
        perm += [69 + 6 * l + c for c in range(3)]
    M6 = np.zeros((6, HID), np.float32)
    for c in range(3):
        M6[c, 32 + c] = 1.0
        M6[3 + c, 35 + c] = 1.0
        for l in range(10):
            M6[c, 38 + 3 * l + c] = 2.0 ** l
            M6[c, 74 + 3 * l + c] = 2.0 ** l
        for l in range(2):
            M6[3 + c, 68 + 3 * l + c] = 2.0 ** l
            M6[3 + c, 104 + 3 * l + c] = 2.0 ** l
    return np.array(perm), M6


_PERM, _M6 = _layouts()


def _permpad(W):
    return jnp.zeros((HID, HID), W.dtype).at[:110].set(W[_PERM])


def kernel(pts, MModel_params, motion_latent, v_template, shapedirs,
           W0, b0, W1, b1, W2, b2, W3, b3, W4, b4, W5, b5, W6, b6, W7, b7,
           Wout, bout):
    B, P, _ = pts.shape
    coeff = MModel_params[0, :150].reshape(150, 1)
    sd2 = shapedirs.reshape(V * 3, 150)
    vt2 = v_template.reshape(V * 3, 1)

    def_flat, shift_flat = pl.pallas_call(
        _blend_body,
        out_shape=[jax.ShapeDtypeStruct((V * 3, 1), jnp.float32),
                   jax.ShapeDtypeStruct((V * 3, 1), jnp.float32)],
    )(vt2, sd2, coeff)

    deformed = def_flat.reshape(V, 3)
    mesh_shift = shift_flat.reshape(V, 3)
    defT = jnp.pad(deformed, ((0, VP - V), (0, 0))).T
    shiftp = jnp.pad(mesh_shift, ((0, VP - V), (0, 0)))

    W0p = _permpad(W0)
    W4a = _permpad(W4[:110])
    W4b = W4[110:238]
    Wm = jnp.stack([W1, W2, W3, W5, W6, W7])
    bs = jnp.stack([b0, b1, b2, b3, b4, b5, b6, b7])
    bout2 = bout.reshape(1, 3)
    M6 = jnp.asarray(_M6)

    cst = lambda *blk: pl.BlockSpec(blk, lambda i: tuple(0 for _ in blk))
    out, feat = pl.pallas_call(
        _main_body,
        grid=(P // PT,),
        in_specs=[
            pl.BlockSpec((PT, 3), lambda i: (i, 0)),
            pl.BlockSpec((PT, 32), lambda i: (i, 0)),
            cst(3, VP),
            cst(VP, 3),
            cst(6, HID),
            cst(HID, HID),
            cst(HID, HID),
            cst(HID, HID),
            cst(6, HID, HID),
            cst(HID, 3),
            cst(8, HID),
            cst(1, 3),
        ],
        out_specs=[pl.BlockSpec((PT, 3), lambda i: (i, 0)),
                   pl.BlockSpec((PT, HID), lambda i: (i, 0))],
        out_shape=[jax.ShapeDtypeStruct((P, 3), jnp.float32),
                   jax.ShapeDtypeStruct((P, HID), jnp.float32)],
        compiler_params=pltpu.CompilerParams(
            dimension_semantics=("parallel",)),
    )(pts[0], motion_latent[0], defT, shiftp, M6,
      W0p, W4a, W4b, Wm, Wout, bs, bout2)

    return out[None], feat[None]


# speedup vs baseline: 3.7348x; 1.2037x over previous
import numpy as np
import jax
import jax.numpy as jnp
from jax.experimental import pallas as pl
from jax.experimental.pallas import tpu as pltpu

V = 5023
VP = 5120
VT = 512
VB = 512
PT = 1024
HID = 128
NT = VP // VT


def _blend_body(vt_ref, sd_ref, co_ref, defT_ref, shift_ref):
    rows = (jax.lax.broadcasted_iota(jnp.int32, (VB, 1), 0)
            + pl.program_id(0) * VB)
    rmask = rows < V
    co = co_ref[...]
    dcols, scols = [], []
    for c in range(3):
        sc = jax.lax.dot_general(sd_ref[:, c, :], co, (((1,), (0,)), ((), ())),
                                 preferred_element_type=jnp.float32)
        d = (vt_ref[:, c:c + 1] + sc) * 4.0
        dcols.append(jnp.where(rmask, d, 0.0))
        scols.append(jnp.where(rmask, -4.0 * sc, 0.0))
    shift_ref[...] = jnp.concatenate(scols, axis=1)
    defT_ref[...] = jnp.concatenate(dcols, axis=1).T


def _main_body(pts_ref, mot_ref, defT_ref, shift_ref, M6_ref, code_ref,
               W0_ref, W4a_ref, W4b_ref, W1_ref, W2_ref, W3_ref,
               W5_ref, W6_ref, W7_ref, Wout_ref, bs_ref, bout_ref,
               out_ref, feat_ref):
    dTf = defT_ref[...]
    vsqf = jnp.sum(dTf * dTf, axis=0, keepdims=True)

    pts = pts_ref[...]
    psq = jnp.sum(pts * pts, axis=1, keepdims=True)

    rmin = jnp.full((PT, 1), jnp.inf, jnp.float32)
    rshift = jnp.zeros((PT, 3), jnp.float32)
    lanef = jax.lax.broadcasted_iota(jnp.int32, (PT, VT), 1).astype(jnp.float32)
    for j in range(NT):
        sl = slice(j * VT, (j + 1) * VT)
        st = shift_ref[sl, :]
        dot = jax.lax.dot_general(pts, dTf[:, sl],
                                  (((1,), (0,)), ((), ())),
                                  preferred_element_type=jnp.float32)
        d2 = jnp.maximum(psq + vsqf[:, sl] - 2.0 * dot, 0.0)
        if j == NT - 1:
            d2 = jnp.where(lanef < float(V - j * VT), d2, jnp.inf)
        tmin = jnp.min(d2, axis=1, keepdims=True)
        cand = jnp.where(d2 == tmin, lanef, float(VT))
        targ = jnp.min(cand, axis=1, keepdims=True)
        onehot = (cand == targ).astype(jnp.float32)
        tshift = jax.lax.dot_general(onehot, st, (((1,), (0,)), ((), ())),
                                     preferred_element_type=jnp.float32)
        upd = tmin < rmin
        rmin = jnp.where(upd, tmin, rmin)
        rshift = jnp.where(upd, tshift, rshift)

    weight = 1.0 / jnp.exp(rmin)
    shifts = rshift * weight

    ps6 = jnp.concatenate([pts, shifts], axis=1)
    lin = jax.lax.dot_general(ps6, M6_ref[...], (((1,), (0,)), ((), ())),
                              preferred_element_type=jnp.float32,
                              precision=jax.lax.Precision.HIGHEST)
    s_all = jnp.sin(lin)
    c_all = jnp.cos(lin)
    code = code_ref[...]
    motp = jnp.concatenate([jnp.zeros((PT, 78), jnp.float32), mot_ref[...],
                            jnp.zeros((PT, HID - 110), jnp.float32)], axis=1)
    feat0 = jnp.where(code == 1, s_all,
             jnp.where(code == 2, c_all,
              jnp.where(code == 0, lin, motp)))

    mm = lambda a, b: jax.lax.dot_general(
        a, b, (((1,), (0,)), ((), ())), preferred_element_type=jnp.float32)

    x = jnp.maximum(mm(feat0[:, 0:110], W0_ref[...]) + bs_ref[0:1, :], 0.0)
    for wref, li in ((W1_ref, 1), (W2_ref, 2), (W3_ref, 3)):
        x = jnp.maximum(mm(x, wref[...]) + bs_ref[li:li + 1, :], 0.0)
    x = jnp.maximum(mm(feat0[:, 0:110], W4a_ref[...]) + mm(x, W4b_ref[...])
                    + bs_ref[4:5, :], 0.0)
    last = x
    for wref, li in ((W5_ref, 5), (W6_ref, 6), (W7_ref, 7)):
        x = mm(x, wref[...]) + bs_ref[li:li + 1, :]
        last = x
        x = jnp.maximum(x, 0.0)
    feat_ref[...] = last
    mlp_out = mm(x, Wout_ref[...]) + bout_ref[...]
    out_ref[...] = pts + shifts + mlp_out


def _consts():
    code = np.full((1, HID), 3, np.int32)
    M6 = np.zeros((6, HID), np.float32)
    for c in range(3):
        code[0, c] = 0
        code[0, 63 + c] = 0
        M6[c, c] = 1.0
        M6[3 + c, 63 + c] = 1.0
        for l in range(10):
            code[0, 3 + 6 * l + c] = 1
            code[0, 6 + 6 * l + c] = 2
            M6[c, 3 + 6 * l + c] = 2.0 ** l
            M6[c, 6 + 6 * l + c] = 2.0 ** l
        for l in range(2):
            code[0, 66 + 6 * l + c] = 1
            code[0, 69 + 6 * l + c] = 2
            M6[3 + c, 66 + 6 * l + c] = 2.0 ** l
            M6[3 + c, 69 + 6 * l + c] = 2.0 ** l
    return code, M6


_CODE, _M6 = _consts()


def kernel(pts, MModel_params, motion_latent, v_template, shapedirs,
           W0, b0, W1, b1, W2, b2, W3, b3, W4, b4, W5, b5, W6, b6, W7, b7,
           Wout, bout):
    B, P, _ = pts.shape
    coeff = MModel_params[0, :150].reshape(150, 1)

    nb = VP // VB
    defT, shiftp = pl.pallas_call(
        _blend_body,
        grid=(nb,),
        in_specs=[
            pl.BlockSpec((VB, 3), lambda i: (i, 0)),
            pl.BlockSpec((VB, 3, 150), lambda i: (i, 0, 0)),
            pl.BlockSpec((150, 1), lambda i: (0, 0)),
        ],
        out_specs=[pl.BlockSpec((3, VB), lambda i: (0, i)),
                   pl.BlockSpec((VB, 3), lambda i: (i, 0))],
        out_shape=[jax.ShapeDtypeStruct((3, VP), jnp.float32),
                   jax.ShapeDtypeStruct((VP, 3), jnp.float32)],
    )(v_template, shapedirs, coeff)

    W4a = W4[:110]
    W4b = W4[110:238]
    bs = jnp.stack([b0, b1, b2, b3, b4, b5, b6, b7])
    bout2 = bout.reshape(1, 3)
    M6 = jnp.asarray(_M6)
    code = jnp.asarray(_CODE)

    cst = lambda *blk: pl.BlockSpec(blk, lambda i: tuple(0 for _ in blk))
    out, feat = pl.pallas_call(
        _main_body,
        grid=(P // PT,),
        in_specs=[
            pl.BlockSpec((PT, 3), lambda i: (i, 0)),
            pl.BlockSpec((PT, 32), lambda i: (i, 0)),
            cst(3, VP),
            cst(VP, 3),
            cst(6, HID),
            cst(1, HID),
            cst(110, HID),
            cst(110, HID),
            cst(HID, HID),
            cst(HID, HID),
            cst(HID, HID),
            cst(HID, HID),
            cst(HID, HID),
            cst(HID, HID),
            cst(HID, HID),
            cst(HID, 3),
            cst(8, HID),
            cst(1, 3),
        ],
        out_specs=[pl.BlockSpec((PT, 3), lambda i: (i, 0)),
                   pl.BlockSpec((PT, HID), lambda i: (i, 0))],
        out_shape=[jax.ShapeDtypeStruct((P, 3), jnp.float32),
                   jax.ShapeDtypeStruct((P, HID), jnp.float32)],
        compiler_params=pltpu.CompilerParams(
            dimension_semantics=("parallel",)),
    )(pts[0], motion_latent[0], defT, shiftp, M6, code,
      W0, W4a, W4b, W1, W2, W3, W5, W6, W7, Wout, bs, bout2)

    return out[None], feat[None]


# speedup vs baseline: 3.7697x; 1.0093x over previous
import numpy as np
import jax
import jax.numpy as jnp
from jax.experimental import pallas as pl
from jax.experimental.pallas import tpu as pltpu

V = 5023
VP = 5120
VT = 512
VB = 1024
PT = 1024
HID = 128
NT = VP // VT


def _blend_body(vt_ref, sd_ref, co_ref, defT_ref, shift_ref):
    rows = (jax.lax.broadcasted_iota(jnp.int32, (VB, 1), 0)
            + pl.program_id(0) * VB)
    rmask = rows < V
    co = co_ref[...]
    dcols, scols = [], []
    for c in range(3):
        sc = jax.lax.dot_general(sd_ref[:, c, :], co, (((1,), (0,)), ((), ())),
                                 preferred_element_type=jnp.float32)
        d = (vt_ref[:, c:c + 1] + sc) * 4.0
        dcols.append(jnp.where(rmask, d, 0.0))
        scols.append(jnp.where(rmask, -4.0 * sc, 0.0))
    shift_ref[...] = jnp.concatenate(scols, axis=1)
    defT_ref[...] = jnp.concatenate(dcols, axis=1).T


def _main_body(pts_ref, mot_ref, defT_ref, shift_ref, M6_ref, code_ref,
               W0_ref, W4a_ref, W4b_ref, W1_ref, W2_ref, W3_ref,
               W5_ref, W6_ref, W7_ref, Wout_ref, bs_ref, bout_ref,
               out_ref, feat_ref):
    dTf = defT_ref[...]
    vsqf = jnp.sum(dTf * dTf, axis=0, keepdims=True)

    pts = pts_ref[...]
    psq = jnp.sum(pts * pts, axis=1, keepdims=True)

    rmin = jnp.full((PT, 1), jnp.inf, jnp.float32)
    rshift = jnp.zeros((PT, 3), jnp.float32)
    lanef = jax.lax.broadcasted_iota(jnp.int32, (PT, VT), 1).astype(jnp.float32)
    for j in range(NT):
        sl = slice(j * VT, (j + 1) * VT)
        st = shift_ref[sl, :]
        dot = jax.lax.dot_general(pts, dTf[:, sl],
                                  (((1,), (0,)), ((), ())),
                                  preferred_element_type=jnp.float32)
        d2 = jnp.maximum(psq + vsqf[:, sl] - 2.0 * dot, 0.0)
        if j == NT - 1:
            d2 = jnp.where(lanef < float(V - j * VT), d2, jnp.inf)
        tmin = jnp.min(d2, axis=1, keepdims=True)
        cand = jnp.where(d2 == tmin, lanef, float(VT))
        targ = jnp.min(cand, axis=1, keepdims=True)
        onehot = (cand == targ).astype(jnp.float32)
        tshift = jax.lax.dot_general(onehot, st, (((1,), (0,)), ((), ())),
                                     preferred_element_type=jnp.float32)
        upd = tmin < rmin
        rmin = jnp.where(upd, tmin, rmin)
        rshift = jnp.where(upd, tshift, rshift)

    weight = 1.0 / jnp.exp(rmin)
    shifts = rshift * weight

    ps6 = jnp.concatenate([pts, shifts], axis=1)
    lin = jax.lax.dot_general(ps6, M6_ref[...], (((1,), (0,)), ((), ())),
                              preferred_element_type=jnp.float32,
                              precision=jax.lax.Precision.HIGHEST)
    s_all = jnp.sin(lin)
    c_all = jnp.cos(lin)
    code = code_ref[...]
    motp = jnp.concatenate([jnp.zeros((PT, 78), jnp.float32), mot_ref[...],
                            jnp.zeros((PT, HID - 110), jnp.float32)], axis=1)
    feat0 = jnp.where(code == 1, s_all,
             jnp.where(code == 2, c_all,
              jnp.where(code == 0, lin, motp)))

    mm = lambda a, b: jax.lax.dot_general(
        a, b, (((1,), (0,)), ((), ())), preferred_element_type=jnp.float32)

    x = jnp.maximum(mm(feat0[:, 0:110], W0_ref[...]) + bs_ref[0:1, :], 0.0)
    for wref, li in ((W1_ref, 1), (W2_ref, 2), (W3_ref, 3)):
        x = jnp.maximum(mm(x, wref[...]) + bs_ref[li:li + 1, :], 0.0)
    x = jnp.maximum(mm(feat0[:, 0:110], W4a_ref[...]) + mm(x, W4b_ref[...])
                    + bs_ref[4:5, :], 0.0)
    last = x
    for wref, li in ((W5_ref, 5), (W6_ref, 6), (W7_ref, 7)):
        x = mm(x, wref[...]) + bs_ref[li:li + 1, :]
        last = x
        x = jnp.maximum(x, 0.0)
    feat_ref[...] = last
    mlp_out = mm(x, Wout_ref[...]) + bout_ref[...]
    out_ref[...] = pts + shifts + mlp_out


def _consts():
    code = np.full((1, HID), 3, np.int32)
    M6 = np.zeros((6, HID), np.float32)
    for c in range(3):
        code[0, c] = 0
        code[0, 63 + c] = 0
        M6[c, c] = 1.0
        M6[3 + c, 63 + c] = 1.0
        for l in range(10):
            code[0, 3 + 6 * l + c] = 1
            code[0, 6 + 6 * l + c] = 2
            M6[c, 3 + 6 * l + c] = 2.0 ** l
            M6[c, 6 + 6 * l + c] = 2.0 ** l
        for l in range(2):
            code[0, 66 + 6 * l + c] = 1
            code[0, 69 + 6 * l + c] = 2
            M6[3 + c, 66 + 6 * l + c] = 2.0 ** l
            M6[3 + c, 69 + 6 * l + c] = 2.0 ** l
    return code, M6


_CODE, _M6 = _consts()


def kernel(pts, MModel_params, motion_latent, v_template, shapedirs,
           W0, b0, W1, b1, W2, b2, W3, b3, W4, b4, W5, b5, W6, b6, W7, b7,
           Wout, bout):
    B, P, _ = pts.shape
    coeff = MModel_params[0, :150].reshape(150, 1)

    nb = VP // VB
    defT, shiftp = pl.pallas_call(
        _blend_body,
        grid=(nb,),
        in_specs=[
            pl.BlockSpec((VB, 3), lambda i: (i, 0)),
            pl.BlockSpec((VB, 3, 150), lambda i: (i, 0, 0)),
            pl.BlockSpec((150, 1), lambda i: (0, 0)),
        ],
        out_specs=[pl.BlockSpec((3, VB), lambda i: (0, i)),
                   pl.BlockSpec((VB, 3), lambda i: (i, 0))],
        out_shape=[jax.ShapeDtypeStruct((3, VP), jnp.float32),
                   jax.ShapeDtypeStruct((VP, 3), jnp.float32)],
    )(v_template, shapedirs, coeff)

    W4a = W4[:110]
    W4b = W4[110:238]
    bs = jnp.stack([b0, b1, b2, b3, b4, b5, b6, b7])
    bout2 = bout.reshape(1, 3)
    M6 = jnp.asarray(_M6)
    code = jnp.asarray(_CODE)

    cst = lambda *blk: pl.BlockSpec(blk, lambda i: tuple(0 for _ in blk))
    out, feat = pl.pallas_call(
        _main_body,
        grid=(P // PT,),
        in_specs=[
            pl.BlockSpec((PT, 3), lambda i: (i, 0)),
            pl.BlockSpec((PT, 32), lambda i: (i, 0)),
            cst(3, VP),
            cst(VP, 3),
            cst(6, HID),
            cst(1, HID),
            cst(110, HID),
            cst(110, HID),
            cst(HID, HID),
            cst(HID, HID),
            cst(HID, HID),
            cst(HID, HID),
            cst(HID, HID),
            cst(HID, HID),
            cst(HID, HID),
            cst(HID, 3),
            cst(8, HID),
            cst(1, 3),
        ],
        out_specs=[pl.BlockSpec((PT, 3), lambda i: (i, 0)),
                   pl.BlockSpec((PT, HID), lambda i: (i, 0))],
        out_shape=[jax.ShapeDtypeStruct((P, 3), jnp.float32),
                   jax.ShapeDtypeStruct((P, HID), jnp.float32)],
        compiler_params=pltpu.CompilerParams(
            dimension_semantics=("parallel",)),
    )(pts[0], motion_latent[0], defT, shiftp, M6, code,
      W0, W4a, W4b, W1, W2, W3, W5, W6, W7, Wout, bs, bout2)

    return out[None], feat[None]
